# baseline (speedup 1.0000x reference)
import functools

import jax
import jax.numpy as jnp
from jax import lax
from jax.experimental import pallas as pl
from jax.experimental.pallas import tpu as pltpu
from jax.experimental.pallas import tpu_sc as plsc

_NC = 2
_NS = 16
_NW = _NC * _NS
_LANES = 16
_NBUF = 4


def _sinusoid_pos(seq_len: int, n_model: int) -> jnp.ndarray:
    pos = jnp.arange(seq_len, dtype=jnp.float32)[:, None]
    exponent = ((jnp.arange(n_model) // 2) * 2).astype(jnp.float32) / n_model
    div = jnp.power(10000.0, exponent)
    ang = pos / div
    even_mask = (jnp.arange(n_model) % 2) == 0
    return jnp.where(even_mask[None, :], jnp.sin(ang), jnp.cos(ang))


@functools.cache
def _build(rows: int, seq_len: int, d: int):
    scale = float(d) ** 0.5
    assert rows % (_NW * seq_len) == 0
    rpw = rows // _NW
    chunk = seq_len
    nchunk = rpw // chunk
    assert nchunk >= _NBUF
    d_units = d // _LANES

    mesh = plsc.VectorSubcoreMesh(core_axis_name="c", subcore_axis_name="s")

    @functools.partial(
        pl.kernel,
        out_type=jax.ShapeDtypeStruct((rows, d), jnp.float32),
        mesh=mesh,
        scratch_types=[
            pltpu.VMEM((seq_len, d), jnp.float32),
            pltpu.VMEM((_NBUF, chunk, d), jnp.float32),
        ] + [pltpu.VMEM((chunk,), jnp.int32)] * _NBUF
          + [pltpu.SemaphoreType.DMA] * (3 * _NBUF),
    )
    def emb(x_hbm, pe_hbm, table_hbm, out_hbm, pe_v, rows_v, *scr):
        idx = scr[:_NBUF]
        isem = scr[_NBUF:2 * _NBUF]
        gsem = scr[2 * _NBUF:3 * _NBUF]
        ssem = scr[3 * _NBUF:4 * _NBUF]
        wid = lax.axis_index("s") * _NC + lax.axis_index("c")
        base = wid * rpw
        pltpu.sync_copy(pe_hbm, pe_v)

        def idx_start(j, s):
            pltpu.async_copy(x_hbm.at[pl.ds(base + j * chunk, chunk)],
                             idx[s], isem[s])

        def idx_wait(s):
            pltpu.make_async_copy(
                x_hbm.at[pl.ds(base, chunk)], idx[s], isem[s]).wait()

        def gather_start(s):
            pltpu.async_copy(table_hbm.at[idx[s]], rows_v.at[s], gsem[s])

        def gather_wait(s):
            pltpu.make_async_copy(
                table_hbm.at[idx[s]], rows_v.at[s], gsem[s]).wait()

        def store_start(j, s):
            pltpu.async_copy(
                rows_v.at[s],
                out_hbm.at[pl.ds(base + j * chunk, chunk)], ssem[s])

        def store_wait(s):
            pltpu.make_async_copy(
                rows_v.at[s],
                out_hbm.at[pl.ds(base, chunk)], ssem[s]).wait()

        for k in range(_NBUF):
            idx_start(k, k)
        idx_wait(0)
        gather_start(0)
        idx_wait(1)
        gather_start(1)

        def body(j, carry):
            for s in range(_NBUF):
                cur = j + s
                la = cur + 2
                sl2 = (s + 2) % _NBUF

                @pl.when(la < nchunk)
                def _():
                    idx_wait(sl2)

                    @pl.when(la >= _NBUF)
                    def _():
                        store_wait(sl2)
                    gather_start(sl2)

                gather_wait(s)

                @pl.when(cur + _NBUF < nchunk)
                def _():
                    idx_start(cur + _NBUF, s)

                def do_row(r, carry2):
                    for c in range(d_units):
                        csl = pl.ds(c * _LANES, _LANES)
                        rows_v[s, r, csl] = (
                            rows_v[s, r, csl] * scale + pe_v[r, csl])
                    return carry2

                lax.fori_loop(0, chunk, do_row, 0, unroll=4)
                store_start(cur, s)
            return carry

        lax.fori_loop(0, nchunk // _NBUF,
                      lambda i, c: body(i * _NBUF, c), 0)
        for s in range(_NBUF):
            store_wait(s)

    return emb


def kernel(x, table):
    b, s = x.shape
    _, d = table.shape
    pe = _sinusoid_pos(s, d)
    emb = _build(b * s, s, d)
    out = emb(x.reshape(-1), pe, table)
    return out.reshape(b, s, d)


# speedup vs baseline: 1.3599x; 1.3599x over previous
import functools

import jax
import jax.numpy as jnp
from jax import lax
from jax.experimental import pallas as pl
from jax.experimental.pallas import tpu as pltpu
from jax.experimental.pallas import tpu_sc as plsc

_NC = 2
_NS = 16
_NW = _NC * _NS
_LANES = 16
_SPC = 2


def _sinusoid_pos(seq_len: int, n_model: int) -> jnp.ndarray:
    pos = jnp.arange(seq_len, dtype=jnp.float32)[:, None]
    exponent = ((jnp.arange(n_model) // 2) * 2).astype(jnp.float32) / n_model
    div = jnp.power(10000.0, exponent)
    ang = pos / div
    even_mask = (jnp.arange(n_model) % 2) == 0
    return jnp.where(even_mask[None, :], jnp.sin(ang), jnp.cos(ang))


@functools.cache
def _build(rows: int, seq_len: int, d: int):
    scale = float(d) ** 0.5
    assert rows % (_NW * seq_len * _SPC) == 0
    rpw = rows // _NW
    chunk = seq_len * _SPC
    nchunk = rpw // chunk
    d_units = d // _LANES

    mesh = plsc.VectorSubcoreMesh(core_axis_name="c", subcore_axis_name="s")

    @functools.partial(
        pl.kernel,
        out_type=jax.ShapeDtypeStruct((rows, d), jnp.float32),
        mesh=mesh,
        scratch_types=[
            pltpu.VMEM((seq_len, d), jnp.float32),
            pltpu.VMEM((chunk,), jnp.int32),
            pltpu.VMEM((chunk,), jnp.int32),
            pltpu.VMEM((2, chunk, d), jnp.float32),
            pltpu.SemaphoreType.DMA,
            pltpu.SemaphoreType.DMA,
            pltpu.SemaphoreType.DMA,
            pltpu.SemaphoreType.DMA,
            pltpu.SemaphoreType.DMA,
            pltpu.SemaphoreType.DMA,
        ],
    )
    def emb(x_hbm, pe_hbm, table_hbm, out_hbm, pe_v, idx0, idx1, rows_v,
            isem0, isem1, gsem0, gsem1, ssem0, ssem1):
        wid = lax.axis_index("s") * _NC + lax.axis_index("c")
        base = wid * rpw
        idx = (idx0, idx1)
        isem = (isem0, isem1)
        gsem = (gsem0, gsem1)
        ssem = (ssem0, ssem1)
        pltpu.sync_copy(pe_hbm, pe_v)

        def idx_start(j, slot):
            pltpu.async_copy(x_hbm.at[pl.ds(base + j * chunk, chunk)],
                             idx[slot], isem[slot])

        def idx_wait(slot):
            pltpu.make_async_copy(
                x_hbm.at[pl.ds(base, chunk)], idx[slot], isem[slot]).wait()

        def gather_start(slot):
            pltpu.async_copy(table_hbm.at[idx[slot]], rows_v.at[slot],
                             gsem[slot])

        def gather_wait(slot):
            pltpu.make_async_copy(
                table_hbm.at[idx[slot]], rows_v.at[slot], gsem[slot]).wait()

        half = 96

        def store_q(j, slot, off, n):
            pltpu.async_copy(
                rows_v.at[slot, pl.ds(off, n)],
                out_hbm.at[pl.ds(base + j * chunk + off, n)],
                ssem[slot])

        def store_wait(slot):
            for n in (half, half, seq_len - half, seq_len - half):
                pltpu.make_async_copy(
                    rows_v.at[slot, pl.ds(0, n)],
                    out_hbm.at[pl.ds(base, n)], ssem[slot]).wait()

        pltpu.sync_copy(x_hbm.at[pl.ds(base, chunk)], idx0)
        idx_start(1, 1)
        gather_start(0)

        def outer(i, carry):
            for b in (0, 1):
                cur = i + b
                nxt = cur + 1

                @pl.when(nxt < nchunk)
                def _():
                    idx_wait(1 - b)

                    @pl.when(cur >= 1)
                    def _():
                        store_wait(1 - b)
                    gather_start(1 - b)

                gather_wait(b)

                @pl.when(cur + 2 < nchunk)
                def _():
                    idx_start(cur + 2, b)

                def do_pos(p, carry2):
                    pvals = [pe_v[p, pl.ds(c * _LANES, _LANES)]
                             for c in range(d_units)]
                    for s in range(_SPC):
                        r = s * seq_len + p
                        for c in range(d_units):
                            sl = pl.ds(c * _LANES, _LANES)
                            rows_v[b, r, sl] = (
                                rows_v[b, r, sl] * scale + pvals[c])
                    return carry2

                lax.fori_loop(0, half, do_pos, 0, unroll=4)
                store_q(cur, b, 0, half)
                store_q(cur, b, seq_len, half)
                lax.fori_loop(half, seq_len, do_pos, 0, unroll=4)
                store_q(cur, b, half, seq_len - half)
                store_q(cur, b, seq_len + half, seq_len - half)
            return carry

        lax.fori_loop(0, nchunk // 2, lambda i, c: outer(i * 2, c), 0)
        store_wait(0)
        store_wait(1)

    return emb


def kernel(x, table):
    b, s = x.shape
    _, d = table.shape
    pe = _sinusoid_pos(s, d)
    emb = _build(b * s, s, d)
    out = emb(x.reshape(-1), pe, table)
    return out.reshape(b, s, d)
